# baseline (speedup 1.0000x reference)
import functools

import jax
import jax.numpy as jnp
from jax import lax
from jax.experimental import pallas as pl
from jax.experimental.pallas import tpu as pltpu
from jax.experimental.pallas import tpu_sc as plsc

_NUM_TAGS = 32
_SHIFT = 128
_L = _NUM_TAGS * _SHIFT
_D = 512
_B = 8
_K = 8
_LBLK = 512
_LANES = 16
_TAGS_PER_WORKER = 8
_CHUNK = _TAGS_PER_WORKER * _SHIFT


def _bcast_kernel(emb_ref, out_ref):
    x = emb_ref[...]
    out_ref[...] = jnp.broadcast_to(x[None], (_B, _LBLK, _D))


_sc_mesh = plsc.VectorSubcoreMesh(core_axis_name="c", subcore_axis_name="s")


@functools.partial(
    pl.kernel,
    out_type=jax.ShapeDtypeStruct((_B * _L,), jnp.int32),
    mesh=_sc_mesh,
    scratch_types=[
        pltpu.VMEM((_B * _K + _LANES,), jnp.int32),
        pltpu.VMEM((_CHUNK,), jnp.int32),
    ],
)
def _sc_mask(tags_hbm, out_hbm, tags_v, chunk_v):
    wid = lax.axis_index("c") * 16 + lax.axis_index("s")
    b = wid // 4
    q = wid % 4
    pltpu.sync_copy(tags_hbm, tags_v.at[pl.ds(0, _B * _K)])
    tv = tags_v[pl.ds(b * _K, _LANES)]
    t0 = q * _TAGS_PER_WORKER
    for ti in range(_TAGS_PER_WORKER):
        t = t0 + ti
        hit = jnp.int32(0)
        for k in range(_K):
            hit = hit | jnp.where(tv[k] == t, jnp.int32(1), jnp.int32(0))
        val = lax.broadcast(hit, (_LANES,))
        for j in range(_SHIFT // _LANES):
            chunk_v[pl.ds(ti * _SHIFT + j * _LANES, _LANES)] = val
    pltpu.sync_copy(chunk_v, out_hbm.at[pl.ds(wid * _CHUNK, _CHUNK)])


def kernel(tags, embedding):
    num_l = _L // _LBLK
    embed = pl.pallas_call(
        _bcast_kernel,
        grid=(num_l,),
        in_specs=[pl.BlockSpec((_LBLK, _D), lambda l: (l, 0))],
        out_specs=pl.BlockSpec((_B, _LBLK, _D), lambda l: (0, l, 0)),
        out_shape=jax.ShapeDtypeStruct((_B, _L, _D), jnp.float32),
    )(embedding)
    mask = _sc_mask(tags.astype(jnp.int32).reshape(_B * _K)).reshape(_B, _L)
    return embed, mask
